# Initial kernel scaffold; baseline (speedup 1.0000x reference)
import jax
import jax.numpy as jnp
from jax.experimental import pallas as pl


def kernel(s_t, adversary_strategy, W_m1, W_m2, W_a1, W_a2, W_p1, W_p2, W_h_a, W_h_s, W_v, w_health, noise):
    raise NotImplementedError("write your pallas kernel here")


# speedup vs baseline: 1.1211x; 1.1211x over previous
import functools

import jax
import jax.numpy as jnp
from jax.experimental import pallas as pl
from jax.experimental.pallas import tpu as pltpu

SD = 64
T = 4096
BLOOM = 16
NB = 16
N0 = T * BLOOM
HORIZON = 8
_INTERP = False

_f32 = jnp.float32


def _preamble_body(s_ref, adv_ref, wm1, wm2, wa1, wa2, wp1, wp2, whs, whc,
                   strat_o, strat_a_o, strat_p_o, ps_o, pi_o, psw_o, h0_o):
    s = s_ref[...]
    adv = adv_ref[...]
    strat = jnp.tanh(adv @ wm1[...] + s @ wm2[...])
    ps = jnp.tanh(strat @ wa1[...] + s @ wa2[...])
    pi = jnp.tanh(strat @ wp1[...] + ps @ wp2[...])
    strat_o[...] = strat
    strat_a_o[...] = strat @ wa1[...]
    strat_p_o[...] = strat @ wp1[...]
    ps_o[...] = ps
    pi_o[...] = pi
    psw_o[...] = ps @ whs[...]
    h0_o[...] = s @ whc[...]


def _round0_body(pi_ref, psw_ref, strat_c, whc, wha, wv, h0_ref, nz_ref,
                 cns_o, cv_o):
    ca = jnp.tanh(pi_ref[...] + nz_ref[...])
    cns = jnp.tanh(psw_ref[...] + ca @ wha[...])
    cns_o[...] = cns
    v = cns @ whc[...] - h0_ref[...]
    vp = (jnp.tanh(cns @ wv[...]) @ strat_c[...]) * ((HORIZON - 1.0) / HORIZON)
    cv_o[...] = v + vp


def _gather_body(first, sel_ref, anc_ref, strat_a, strat_p, wa2, wp2, whs,
                 cns_ref, psw_o, pi_o, anc_o, cst_ref):
    def step(i, carry):
        for u in range(8):
            k = i * 8 + u
            idx = sel_ref[k]
            cst_ref[pl.ds(k, 1), :] = cns_ref[pl.ds(idx, 1), :]
            if first:
                anc_o[k] = idx
            else:
                anc_o[k] = anc_ref[idx % T]
        return carry

    jax.lax.fori_loop(0, T // 8, step, 0)
    cst = cst_ref[...]
    ps_b = jnp.tanh(strat_a[...] + cst @ wa2[...])
    pi_o[...] = jnp.tanh(strat_p[...] + ps_b @ wp2[...])
    psw_o[...] = ps_b @ whs[...]


def _branch_body(scale, pi_ref, psw_ref, nz_ref, strat_c, whc, wha, wv,
                 h0_ref, ns_o, cv_o):
    a = jnp.tanh(pi_ref[...] + nz_ref[0])
    ns = jnp.tanh(psw_ref[...] + a @ wha[...])
    ns_o[...] = ns
    v = ns @ whc[...] - h0_ref[...]
    vp = (jnp.tanh(ns @ wv[...]) @ strat_c[...]) * scale
    cv_o[...] = v + vp


def _final_body(cv_ref, anc_ref, pi_ref, nz_ref, out_o):
    x = cv_ref[...]
    m = jnp.max(x)
    ii = jax.lax.broadcasted_iota(jnp.int32, (512, 128), 0) * 128 + \
        jax.lax.broadcasted_iota(jnp.int32, (512, 128), 1)
    flat = jnp.min(jnp.where(x == m, ii, jnp.int32(2 ** 30)))
    g = anc_ref[flat % T]
    out_o[...] = jnp.tanh(pi_ref[...] + nz_ref[pl.ds(g, 1), :])


def kernel(s_t, adversary_strategy, W_m1, W_m2, W_a1, W_a2, W_p1, W_p2,
           W_h_a, W_h_s, W_v, w_health, noise):
    call = functools.partial(pl.pallas_call, interpret=_INTERP)
    f32 = _f32
    s2 = s_t.reshape(1, SD)
    adv2 = adversary_strategy.reshape(1, SD)
    whc = w_health.reshape(SD, 1)

    vec = jax.ShapeDtypeStruct((1, SD), f32)
    strat, strat_a, strat_p, ps, pi, psw, h0 = call(
        _preamble_body,
        out_shape=[vec, vec, vec, vec, vec, vec,
                   jax.ShapeDtypeStruct((1, 1), f32)],
    )(s2, adv2, W_m1, W_m2, W_a1, W_a2, W_p1, W_p2, W_h_s, whc)
    strat_c = strat.reshape(SD, 1)

    blk = pl.BlockSpec((T, SD), lambda j: (j, 0))
    rep = pl.BlockSpec((1, SD), lambda j: (0, 0))
    rep_c = pl.BlockSpec((SD, 1), lambda j: (0, 0))
    rep_m = pl.BlockSpec((SD, SD), lambda j: (0, 0))
    rep_s = pl.BlockSpec((1, 1), lambda j: (0, 0))
    cns0, cv0 = call(
        _round0_body,
        grid=(16,),
        in_specs=[rep, rep, rep_c, rep_c, rep_m, rep_m, rep_s,
                  pl.BlockSpec((T, SD), lambda j: (j, 0))],
        out_specs=[blk, pl.BlockSpec((T, 1), lambda j: (j, 0))],
        out_shape=[jax.ShapeDtypeStruct((N0, SD), f32),
                   jax.ShapeDtypeStruct((N0, 1), f32)],
    )(pi, psw, strat_c, whc, W_h_a, W_v, h0, noise)

    noise16 = noise[:NB].reshape(NB, 1, SD)
    anc = jnp.zeros((T,), jnp.int32)
    sel = jax.lax.top_k(cv0.reshape(-1), T)[1].astype(jnp.int32)
    cns = cns0
    for rnd in (1, 2):
        smem = pl.BlockSpec(memory_space=pltpu.SMEM)
        psw_b, pi_b, anc = call(
            functools.partial(_gather_body, rnd == 1),
            in_specs=[smem, smem]
            + [pl.BlockSpec((1, SD), lambda: (0, 0)),
               pl.BlockSpec((1, SD), lambda: (0, 0)),
               pl.BlockSpec((SD, SD), lambda: (0, 0)),
               pl.BlockSpec((SD, SD), lambda: (0, 0)),
               pl.BlockSpec((SD, SD), lambda: (0, 0)),
               pl.BlockSpec((N0, SD), lambda: (0, 0))],
            out_specs=[pl.BlockSpec((T, SD), lambda: (0, 0)),
                       pl.BlockSpec((T, SD), lambda: (0, 0)),
                       pl.BlockSpec(memory_space=pltpu.SMEM)],
            out_shape=[jax.ShapeDtypeStruct((T, SD), f32),
                       jax.ShapeDtypeStruct((T, SD), f32),
                       jax.ShapeDtypeStruct((T,), jnp.int32)],
            scratch_shapes=[pltpu.VMEM((T, SD), f32)],
        )(sel, anc, strat_a, strat_p, W_a2, W_p2, W_h_s, cns)

        scale = (HORIZON - 1.0 - rnd) / HORIZON
        full = pl.BlockSpec((T, SD), lambda j: (0, 0))
        ns, cv = call(
            functools.partial(_branch_body, scale),
            grid=(NB,),
            in_specs=[full, full,
                      pl.BlockSpec((1, 1, SD), lambda j: (j, 0, 0)),
                      rep_c, rep_c, rep_m, rep_m, rep_s],
            out_specs=[blk, pl.BlockSpec((T, 1), lambda j: (j, 0))],
            out_shape=[jax.ShapeDtypeStruct((N0, SD), f32),
                       jax.ShapeDtypeStruct((N0, 1), f32)],
        )(pi_b, psw_b, noise16, strat_c, whc, W_h_a, W_v, h0)
        if rnd < 2:
            sel = jax.lax.top_k(cv.reshape(-1), T)[1].astype(jnp.int32)
            cns = ns

    out = call(
        _final_body,
        in_specs=[pl.BlockSpec((512, 128), lambda: (0, 0)),
                  pl.BlockSpec(memory_space=pltpu.SMEM),
                  pl.BlockSpec((1, SD), lambda: (0, 0)),
                  pl.BlockSpec((N0, SD), lambda: (0, 0))],
        out_specs=pl.BlockSpec((1, SD), lambda: (0, 0)),
        out_shape=jax.ShapeDtypeStruct((1, SD), f32),
    )(cv.reshape(512, 128), anc, pi, noise)
    return out.reshape(SD)


# speedup vs baseline: 1.1751x; 1.0482x over previous
import dataclasses
import functools

import jax
import jax.numpy as jnp
from jax.experimental import pallas as pl
from jax.experimental.pallas import tpu as pltpu
from jax.experimental.pallas import tpu_sc as plsc

SD = 64
T = 4096
BLOOM = 16
NB = 16
N0 = T * BLOOM
HORIZON = 8
_INTERP = False

@functools.cache
def _sc_mesh():
    return plsc.VectorSubcoreMesh(core_axis_name="c", subcore_axis_name="s")


@functools.cache
def _sc_params():
    cp = pltpu.CompilerParams(use_tc_tiling_on_sc=False)
    if "needs_layout_passes" in pltpu.CompilerParams.__dataclass_fields__:
        cp = dataclasses.replace(cp, needs_layout_passes=False)
    return cp


def _preamble_body(s_ref, adv_ref, wm1, wm2, wa1, wa2, wp1, wp2, whs, whc,
                   strat_o, strat_a_o, strat_p_o, pi_o, psw_o, h0_o):
    s = s_ref[...]
    adv = adv_ref[...]
    strat = jnp.tanh(adv @ wm1[...] + s @ wm2[...])
    ps = jnp.tanh(strat @ wa1[...] + s @ wa2[...])
    pi = jnp.tanh(strat @ wp1[...] + ps @ wp2[...])
    strat_o[...] = strat
    strat_a_o[...] = strat @ wa1[...]
    strat_p_o[...] = strat @ wp1[...]
    pi_o[...] = pi
    psw_o[...] = ps @ whs[...]
    h0_o[...] = s @ whc[...]


def _round0_body(pi_ref, psw_ref, strat_c, whc, wha, wv, h0_ref, nz_ref,
                 cns_o, cv_o):
    ca = jnp.tanh(pi_ref[...] + nz_ref[...])
    cns = jnp.tanh(psw_ref[...] + ca @ wha[...])
    cns_o[...] = cns
    v = cns @ whc[...] - h0_ref[...]
    vp = (jnp.tanh(cns @ wv[...]) @ strat_c[...]) * ((HORIZON - 1.0) / HORIZON)
    cv_o[...] = v + vp


def _thresh_body(cv_ref, keys_o, pref_o, meta_o):
    v = cv_ref[...]
    s = jax.lax.bitcast_convert_type(v, jnp.int32)
    key = s ^ ((s >> 31) & jnp.int32(0x7FFFFFFF))
    keys_o[...] = key
    msb = jnp.int32(-2147483648)

    def bit(i, pu):
        cand_u = pu | (jnp.int32(1) << (31 - i))
        cand_s = cand_u ^ msb
        cnt = jnp.sum((key >= cand_s).astype(jnp.int32))
        return jnp.where(cnt >= T, cand_u, pu)

    pu = jax.lax.fori_loop(0, 32, bit, jnp.int32(0))
    t = pu ^ msb
    gt = key > t
    eq = key == t
    rem = T - jnp.sum(gt.astype(jnp.int32))
    ii = (jax.lax.broadcasted_iota(jnp.int32, (512, 128), 0) * 128
          + jax.lax.broadcasted_iota(jnp.int32, (512, 128), 1))

    def bit2(i, p0):
        cand = p0 | (jnp.int32(1) << (15 - i))
        c = jnp.sum((eq & (ii < cand)).astype(jnp.int32))
        return jnp.where(c < rem, cand, p0)

    p0 = jax.lax.fori_loop(0, 16, bit2, jnp.int32(0))
    mask = gt | (eq & (ii <= p0))
    mf = mask.astype(jnp.float32)
    sel16 = (jax.lax.broadcasted_iota(jnp.int32, (128, 8), 0) // 16
             == jax.lax.broadcasted_iota(jnp.int32, (128, 8), 1))
    cc = jnp.dot(mf, sel16.astype(jnp.float32))
    upper = (jax.lax.broadcasted_iota(jnp.int32, (8, 8), 0)
             < jax.lax.broadcasted_iota(jnp.int32, (8, 8), 1))
    rowpref = jnp.dot(cc, upper.astype(jnp.float32))
    rt = jnp.dot(cc, jnp.ones((8, 1), jnp.float32))
    lower = (jax.lax.broadcasted_iota(jnp.int32, (512, 512), 1)
             < jax.lax.broadcasted_iota(jnp.int32, (512, 512), 0))
    rtp = jnp.dot(lower.astype(jnp.float32), rt)
    pref_o[...] = (rowpref + rtp).astype(jnp.int32)
    meta_o[0] = t
    meta_o[1] = p0
    for q in range(2, 16):
        meta_o[q] = jnp.int32(0)


def _sc_compact(first):
    def body(keys_hbm, meta_hbm, pref_hbm, anc_hbm, selpad_o, ancpad_o,
             kv, pv, mv, av, sb, ab):
        tile = jax.lax.axis_index("c") * 16 + jax.lax.axis_index("s")
        base = tile * 2048
        pltpu.sync_copy(keys_hbm.at[pl.ds(base, 2048)], kv)
        pltpu.sync_copy(pref_hbm.at[pl.ds(tile * 128, 128)],
                        pv.at[pl.ds(0, 128)])
        pltpu.sync_copy(meta_hbm, mv)
        if not first:
            pltpu.sync_copy(anc_hbm, av)
        mvv = mv[...]
        t = mvv[0]
        p0 = mvv[1]
        lane = jax.lax.iota(jnp.int32, 16)
        zero = lane - lane

        @pl.loop(0, 260)
        def _(c):
            sb[pl.ds(c * 16, 16)] = zero
            ab[pl.ds(c * 16, 16)] = zero

        @pl.loop(0, 128)
        def _(c):
            off = c * 16
            k16 = kv[pl.ds(off, 16)]
            gidx = lane + (base + off)
            m = (k16 > t) | ((k16 == t) & (gidx <= p0))
            mi = m.astype(jnp.int32)
            exc = plsc.cumsum(mi) - mi
            pvc = pv[pl.ds(c, 16)][0]
            dst = jnp.where(m, pvc + exc, T + lane)
            plsc.store_scatter(sb, [dst], gidx, mask=m)
            if first:
                av_ = gidx
            else:
                av_ = plsc.load_gather(av, [gidx & (T - 1)])
            plsc.store_scatter(ab, [dst], av_, mask=m)

        pltpu.sync_copy(sb, selpad_o.at[tile])
        pltpu.sync_copy(ab, ancpad_o.at[tile])

    return body


def _sc_gather_body(selpad_hbm, ancpad_hbm, cns_hbm, cst_o, anc_o,
                    sp, ap, accs, acca, rows):
    tile = jax.lax.axis_index("c") * 16 + jax.lax.axis_index("s")
    off = tile * 128
    pltpu.sync_copy(selpad_hbm.at[:, pl.ds(off, 128)], sp)
    pltpu.sync_copy(ancpad_hbm.at[:, pl.ds(off, 128)], ap)

    @pl.loop(0, 8)
    def _(c):
        o16 = c * 16
        s = sp[0, pl.ds(o16, 16)]
        a = ap[0, pl.ds(o16, 16)]
        for r in range(1, 32):
            s = s + sp[r, pl.ds(o16, 16)]
            a = a + ap[r, pl.ds(o16, 16)]
        accs[pl.ds(o16, 16)] = s
        acca[pl.ds(o16, 16)] = a

    pltpu.sync_copy(cns_hbm.at[accs], rows)
    pltpu.sync_copy(rows, cst_o.at[pl.ds(off, 128)])
    pltpu.sync_copy(acca, anc_o.at[pl.ds(off, 128)])


def _dense_body(cst_ref, strat_a, strat_p, wa2, wp2, whs, psw_o, pi_o):
    ps_b = jnp.tanh(strat_a[...] + cst_ref[...] @ wa2[...])
    pi_o[...] = jnp.tanh(strat_p[...] + ps_b @ wp2[...])
    psw_o[...] = ps_b @ whs[...]


def _branch_body(scale, last, pi_ref, psw_ref, nz_ref, strat_c, whc, wha, wv,
                 h0_ref, *outs):
    a = jnp.tanh(pi_ref[...] + nz_ref[0])
    ns = jnp.tanh(psw_ref[...] + a @ wha[...])
    v = ns @ whc[...] - h0_ref[...]
    vp = (jnp.tanh(ns @ wv[...]) @ strat_c[...]) * scale
    if last:
        outs[0][...] = v + vp
    else:
        outs[0][...] = ns
        outs[1][...] = v + vp


def _final_body(cv_ref, anc_ref, pi_ref, nz_ref, out_o):
    x = cv_ref[...]
    m = jnp.max(x)
    ii = (jax.lax.broadcasted_iota(jnp.int32, (512, 128), 0) * 128
          + jax.lax.broadcasted_iota(jnp.int32, (512, 128), 1))
    flat = jnp.min(jnp.where(x == m, ii, jnp.int32(2 ** 30)))
    g = anc_ref[flat % T]
    out_o[...] = jnp.tanh(pi_ref[...] + nz_ref[pl.ds(g, 1), :])


def _select(call, cv, anc, first):
    keys, pref, meta = call(
        _thresh_body,
        in_specs=[pl.BlockSpec((512, 128), lambda: (0, 0))],
        out_specs=[pl.BlockSpec((512, 128), lambda: (0, 0)),
                   pl.BlockSpec((512, 8), lambda: (0, 0)),
                   pl.BlockSpec(memory_space=pltpu.SMEM)],
        out_shape=[jax.ShapeDtypeStruct((512, 128), jnp.int32),
                   jax.ShapeDtypeStruct((512, 8), jnp.int32),
                   jax.ShapeDtypeStruct((16,), jnp.int32)],
    )(cv.reshape(512, 128))

    i32 = jnp.int32
    selpad, ancpad = pl.kernel(
        _sc_compact(first),
        out_type=[jax.ShapeDtypeStruct((32, 4160), i32),
                  jax.ShapeDtypeStruct((32, 4160), i32)],
        mesh=_sc_mesh(),
        compiler_params=_sc_params(),
        scratch_types=[pltpu.VMEM((2048,), i32),
                       pltpu.VMEM((144,), i32),
                       pltpu.VMEM((16,), i32),
                       pltpu.VMEM((T,), i32),
                       pltpu.VMEM((4160,), i32),
                       pltpu.VMEM((4160,), i32)],
        interpret=_INTERP,
    )(keys.reshape(N0), meta, pref.reshape(T), anc)
    return selpad, ancpad


def _sc_gather(selpad, ancpad, cns):
    return pl.kernel(
        _sc_gather_body,
        out_type=[jax.ShapeDtypeStruct((T, SD), jnp.float32),
                  jax.ShapeDtypeStruct((T,), jnp.int32)],
        mesh=_sc_mesh(),
        compiler_params=_sc_params(),
        scratch_types=[pltpu.VMEM((32, 128), jnp.int32),
                       pltpu.VMEM((32, 128), jnp.int32),
                       pltpu.VMEM((128,), jnp.int32),
                       pltpu.VMEM((128,), jnp.int32),
                       pltpu.VMEM((128, SD), jnp.float32)],
        interpret=_INTERP,
    )(selpad, ancpad, cns)


def kernel(s_t, adversary_strategy, W_m1, W_m2, W_a1, W_a2, W_p1, W_p2,
           W_h_a, W_h_s, W_v, w_health, noise):
    call = functools.partial(pl.pallas_call, interpret=_INTERP)
    f32 = jnp.float32
    s2 = s_t.reshape(1, SD)
    adv2 = adversary_strategy.reshape(1, SD)
    whc = w_health.reshape(SD, 1)

    vec = jax.ShapeDtypeStruct((1, SD), f32)
    strat, strat_a, strat_p, pi, psw, h0 = call(
        _preamble_body,
        out_shape=[vec, vec, vec, vec, vec,
                   jax.ShapeDtypeStruct((1, 1), f32)],
    )(s2, adv2, W_m1, W_m2, W_a1, W_a2, W_p1, W_p2, W_h_s, whc)
    strat_c = strat.reshape(SD, 1)

    blk = pl.BlockSpec((T, SD), lambda j: (j, 0))
    rep = pl.BlockSpec((1, SD), lambda j: (0, 0))
    rep_c = pl.BlockSpec((SD, 1), lambda j: (0, 0))
    rep_m = pl.BlockSpec((SD, SD), lambda j: (0, 0))
    rep_s = pl.BlockSpec((1, 1), lambda j: (0, 0))
    cns0, cv0 = call(
        _round0_body,
        grid=(16,),
        in_specs=[rep, rep, rep_c, rep_c, rep_m, rep_m, rep_s,
                  pl.BlockSpec((T, SD), lambda j: (j, 0))],
        out_specs=[blk, pl.BlockSpec((T, 1), lambda j: (j, 0))],
        out_shape=[jax.ShapeDtypeStruct((N0, SD), f32),
                   jax.ShapeDtypeStruct((N0, 1), f32)],
    )(pi, psw, strat_c, whc, W_h_a, W_v, h0, noise)

    noise16 = noise[:NB].reshape(NB, 1, SD)
    anc = jnp.zeros((T,), jnp.int32)
    cv, cns = cv0, cns0
    for rnd in (1, 2):
        selpad, ancpad = _select(call, cv, anc, rnd == 1)
        cst, anc = _sc_gather(selpad, ancpad, cns)

        full0 = pl.BlockSpec((T, SD), lambda: (0, 0))
        psw_b, pi_b = call(
            _dense_body,
            in_specs=[full0,
                      pl.BlockSpec((1, SD), lambda: (0, 0)),
                      pl.BlockSpec((1, SD), lambda: (0, 0)),
                      pl.BlockSpec((SD, SD), lambda: (0, 0)),
                      pl.BlockSpec((SD, SD), lambda: (0, 0)),
                      pl.BlockSpec((SD, SD), lambda: (0, 0))],
            out_specs=[full0, full0],
            out_shape=[jax.ShapeDtypeStruct((T, SD), f32),
                       jax.ShapeDtypeStruct((T, SD), f32)],
        )(cst, strat_a, strat_p, W_a2, W_p2, W_h_s)

        scale = (HORIZON - 1.0 - rnd) / HORIZON
        last = rnd == 2
        full = pl.BlockSpec((T, SD), lambda j: (0, 0))
        cv_spec = pl.BlockSpec((T, 1), lambda j: (j, 0))
        cv_shape = jax.ShapeDtypeStruct((N0, 1), f32)
        outs = call(
            functools.partial(_branch_body, scale, last),
            grid=(NB,),
            in_specs=[full, full,
                      pl.BlockSpec((1, 1, SD), lambda j: (j, 0, 0)),
                      rep_c, rep_c, rep_m, rep_m, rep_s],
            out_specs=[cv_spec] if last else [blk, cv_spec],
            out_shape=[cv_shape] if last
            else [jax.ShapeDtypeStruct((N0, SD), f32), cv_shape],
        )(pi_b, psw_b, noise16, strat_c, whc, W_h_a, W_v, h0)
        if last:
            cv = outs if isinstance(outs, jax.Array) else outs[0]
        else:
            cns, cv = outs

    out = call(
        _final_body,
        in_specs=[pl.BlockSpec((512, 128), lambda: (0, 0)),
                  pl.BlockSpec(memory_space=pltpu.SMEM),
                  pl.BlockSpec((1, SD), lambda: (0, 0)),
                  pl.BlockSpec((N0, SD), lambda: (0, 0))],
        out_specs=pl.BlockSpec((1, SD), lambda: (0, 0)),
        out_shape=jax.ShapeDtypeStruct((1, SD), f32),
    )(cv.reshape(512, 128), anc, pi, noise)
    return out.reshape(SD)
